# Initial kernel scaffold; baseline (speedup 1.0000x reference)
import math

import jax
import jax.numpy as jnp
from jax.experimental import pallas as pl
from jax.experimental.pallas import tpu as pltpu

_THETA = 10000.0
_D = 128
_LN_THETA = math.log(_THETA)

_BH_BLK = 16
_S_BLK = 512


def _rope_kernel(pos_ref, x_ref, o_ref):
    x = x_ref[...]
    pos = pos_ref[0].astype(jnp.float32)

    lane = jax.lax.broadcasted_iota(jnp.int32, (_S_BLK, _D), 1)
    pair = (lane // 2).astype(jnp.float32)
    inv_theta = jnp.exp(pair * (-2.0 * _LN_THETA / _D))
    freqs = pos * inv_theta
    cos_e = jnp.cos(freqs)
    sign = jnp.where(lane % 2 == 0, -1.0, 1.0)
    sin_m = jnp.sin(freqs) * sign

    even = (lane % 2 == 0)[None]
    x_sw = jnp.where(even, pltpu.roll(x, -1, axis=2), pltpu.roll(x, 1, axis=2))
    o_ref[...] = x * cos_e[None] + x_sw * sin_m[None]


def kernel(x, token_positions):
    b, h, s, d = x.shape
    bh = b * h
    xr = x.reshape(bh, s, d)
    pos3 = token_positions.reshape(s // _S_BLK, _S_BLK, 1)

    out = pl.pallas_call(
        _rope_kernel,
        grid=(s // _S_BLK, bh // _BH_BLK),
        in_specs=[
            pl.BlockSpec((1, _S_BLK, 1), lambda i, j: (i, 0, 0)),
            pl.BlockSpec((_BH_BLK, _S_BLK, d), lambda i, j: (j, i, 0)),
        ],
        out_specs=pl.BlockSpec((_BH_BLK, _S_BLK, d), lambda i, j: (j, i, 0)),
        out_shape=jax.ShapeDtypeStruct((bh, s, d), x.dtype),
    )(pos3, xr)
    return out.reshape(b, h, s, d)


# speedup vs baseline: 3.5775x; 3.5775x over previous
import math

import jax
import jax.numpy as jnp
from jax.experimental import pallas as pl
from jax.experimental.pallas import tpu as pltpu

_THETA = 10000.0
_D = 128
_LN_THETA = math.log(_THETA)

_BH_BLK = 16
_S_BLK = 512


def _rope_kernel(pos_ref, x_ref, o_ref, cos_ref, sin_ref):
    lane = jax.lax.broadcasted_iota(jnp.int32, (_S_BLK, _D), 1)

    @pl.when(pl.program_id(1) == 0)
    def _():
        pos = pos_ref[0].astype(jnp.float32)
        pair = (lane // 2).astype(jnp.float32)
        inv_theta = jnp.exp(pair * (-2.0 * _LN_THETA / _D))
        freqs = pos * inv_theta
        sign = jnp.where(lane % 2 == 0, -1.0, 1.0)
        cos_ref[...] = jnp.cos(freqs)
        sin_ref[...] = jnp.sin(freqs) * sign

    x = x_ref[...]
    even = (lane % 2 == 0)[None]
    x_sw = jnp.where(even, pltpu.roll(x, _D - 1, axis=2), pltpu.roll(x, 1, axis=2))
    o_ref[...] = x * cos_ref[...][None] + x_sw * sin_ref[...][None]


def kernel(x, token_positions):
    b, h, s, d = x.shape
    bh = b * h
    xr = x.reshape(bh, s, d)
    pos3 = token_positions.reshape(s // _S_BLK, _S_BLK, 1)

    out = pl.pallas_call(
        _rope_kernel,
        grid=(s // _S_BLK, bh // _BH_BLK),
        in_specs=[
            pl.BlockSpec((1, _S_BLK, 1), lambda i, j: (i, 0, 0)),
            pl.BlockSpec((_BH_BLK, _S_BLK, d), lambda i, j: (j, i, 0)),
        ],
        out_specs=pl.BlockSpec((_BH_BLK, _S_BLK, d), lambda i, j: (j, i, 0)),
        out_shape=jax.ShapeDtypeStruct((bh, s, d), x.dtype),
        scratch_shapes=[
            pltpu.VMEM((_S_BLK, _D), jnp.float32),
            pltpu.VMEM((_S_BLK, _D), jnp.float32),
        ],
    )(pos3, xr)
    return out.reshape(b, h, s, d)


# speedup vs baseline: 3.9869x; 1.1144x over previous
import math

import jax
import jax.numpy as jnp
from jax.experimental import pallas as pl
from jax.experimental.pallas import tpu as pltpu

_THETA = 10000.0
_D = 128
_LN_THETA = math.log(_THETA)

_BH_BLK = 16
_S_BLK = 512


def _rope_kernel(pos_ref, x_ref, o_ref, cos_ref, sin_ref):
    lane = jax.lax.broadcasted_iota(jnp.int32, (_S_BLK, _D), 1)

    @pl.when(pl.program_id(1) == 0)
    def _():
        pos = pos_ref[0].astype(jnp.float32)
        pair = (lane // 2).astype(jnp.float32)
        inv_theta = jnp.exp(pair * (-2.0 * _LN_THETA / _D))
        freqs = pos * inv_theta
        sign = jnp.where(lane % 2 == 0, -1.0, 1.0)
        cos_ref[...] = jnp.cos(freqs)
        sin_ref[...] = jnp.sin(freqs) * sign

    x = x_ref[...]
    idx = jax.lax.broadcasted_iota(jnp.int32, x.shape, 2) ^ 1
    x_sw = jnp.take_along_axis(x, idx, axis=2)
    o_ref[...] = x * cos_ref[...][None] + x_sw * sin_ref[...][None]


def kernel(x, token_positions):
    b, h, s, d = x.shape
    bh = b * h
    xr = x.reshape(bh, s, d)
    pos3 = token_positions.reshape(s // _S_BLK, _S_BLK, 1)

    out = pl.pallas_call(
        _rope_kernel,
        grid=(s // _S_BLK, bh // _BH_BLK),
        in_specs=[
            pl.BlockSpec((1, _S_BLK, 1), lambda i, j: (i, 0, 0)),
            pl.BlockSpec((_BH_BLK, _S_BLK, d), lambda i, j: (j, i, 0)),
        ],
        out_specs=pl.BlockSpec((_BH_BLK, _S_BLK, d), lambda i, j: (j, i, 0)),
        out_shape=jax.ShapeDtypeStruct((bh, s, d), x.dtype),
        scratch_shapes=[
            pltpu.VMEM((_S_BLK, _D), jnp.float32),
            pltpu.VMEM((_S_BLK, _D), jnp.float32),
        ],
    )(pos3, xr)
    return out.reshape(b, h, s, d)


# speedup vs baseline: 4.1953x; 1.0523x over previous
import math

import jax
import jax.numpy as jnp
from jax.experimental import pallas as pl
from jax.experimental.pallas import tpu as pltpu

_THETA = 10000.0
_D = 128
_LN_THETA = math.log(_THETA)

_BH_BLK = 32
_S_BLK = 512


def _rope_kernel(pos_ref, x_ref, o_ref, cos_ref, sin_ref):
    lane = jax.lax.broadcasted_iota(jnp.int32, (_S_BLK, _D), 1)

    @pl.when(pl.program_id(1) == 0)
    def _():
        pos = pos_ref[0].astype(jnp.float32)
        pair = (lane // 2).astype(jnp.float32)
        inv_theta = jnp.exp(pair * (-2.0 * _LN_THETA / _D))
        freqs = pos * inv_theta
        sign = jnp.where(lane % 2 == 0, -1.0, 1.0)
        cos_ref[...] = jnp.cos(freqs)
        sin_ref[...] = jnp.sin(freqs) * sign

    x = x_ref[...]
    idx = jax.lax.broadcasted_iota(jnp.int32, x.shape, 2) ^ 1
    x_sw = jnp.take_along_axis(x, idx, axis=2)
    o_ref[...] = x * cos_ref[...][None] + x_sw * sin_ref[...][None]


def kernel(x, token_positions):
    b, h, s, d = x.shape
    bh = b * h
    xr = x.reshape(bh, s, d)
    pos3 = token_positions.reshape(s // _S_BLK, _S_BLK, 1)

    out = pl.pallas_call(
        _rope_kernel,
        grid=(s // _S_BLK, bh // _BH_BLK),
        in_specs=[
            pl.BlockSpec((1, _S_BLK, 1), lambda i, j: (i, 0, 0)),
            pl.BlockSpec((_BH_BLK, _S_BLK, d), lambda i, j: (j, i, 0)),
        ],
        out_specs=pl.BlockSpec((_BH_BLK, _S_BLK, d), lambda i, j: (j, i, 0)),
        out_shape=jax.ShapeDtypeStruct((bh, s, d), x.dtype),
        scratch_shapes=[
            pltpu.VMEM((_S_BLK, _D), jnp.float32),
            pltpu.VMEM((_S_BLK, _D), jnp.float32),
        ],
    )(pos3, xr)
    return out.reshape(b, h, s, d)


# speedup vs baseline: 4.2020x; 1.0016x over previous
import math

import jax
import jax.numpy as jnp
from jax.experimental import pallas as pl
from jax.experimental.pallas import tpu as pltpu

_THETA = 10000.0
_D = 128
_LN_THETA = math.log(_THETA)

_BH_BLK = 32
_S_BLK = 512


def _rope_kernel(pos_ref, x_ref, o_ref, cos_ref, sin_ref):
    lane = jax.lax.broadcasted_iota(jnp.int32, (_S_BLK, _D), 1)

    @pl.when(pl.program_id(1) == 0)
    def _():
        pos = pos_ref[0].astype(jnp.float32)
        pair = (lane // 2).astype(jnp.float32)
        inv_theta = jnp.exp(pair * (-2.0 * _LN_THETA / _D))
        freqs = pos * inv_theta
        sign = jnp.where(lane % 2 == 0, -1.0, 1.0)
        cos_ref[...] = jnp.cos(freqs)
        sin_ref[...] = jnp.sin(freqs) * sign

    x = x_ref[...]
    idx = jax.lax.broadcasted_iota(jnp.int32, x.shape, 2) ^ 1
    x_sw = jnp.take_along_axis(x, idx, axis=2)
    o_ref[...] = x * cos_ref[...][None] + x_sw * sin_ref[...][None]


def kernel(x, token_positions):
    b, h, s, d = x.shape
    bh = b * h
    xr = x.reshape(bh, s, d)
    pos3 = token_positions.reshape(s // _S_BLK, _S_BLK, 1)

    out = pl.pallas_call(
        _rope_kernel,
        grid=(s // _S_BLK, bh // _BH_BLK),
        in_specs=[
            pl.BlockSpec((1, _S_BLK, 1), lambda i, j: (i, 0, 0)),
            pl.BlockSpec((_BH_BLK, _S_BLK, d), lambda i, j: (j, i, 0)),
        ],
        out_specs=pl.BlockSpec((_BH_BLK, _S_BLK, d), lambda i, j: (j, i, 0)),
        out_shape=jax.ShapeDtypeStruct((bh, s, d), x.dtype),
        scratch_shapes=[
            pltpu.VMEM((_S_BLK, _D), jnp.float32),
            pltpu.VMEM((_S_BLK, _D), jnp.float32),
        ],
        compiler_params=pltpu.CompilerParams(
            dimension_semantics=("parallel", "arbitrary"),
        ),
    )(pos3, xr)
    return out.reshape(b, h, s, d)
